# baseline (speedup 1.0000x reference)
import jax
import jax.numpy as jnp
from jax import lax
from jax.experimental import pallas as pl
from jax.experimental.pallas import tpu as pltpu
from jax.experimental.pallas import tpu_sc as plsc

N_NODES = 10000
D_FEAT = 128
N_EDGES = 320000

NC = 2
NS = 16
K = 112
S = (N_EDGES + NC * NS * K - 1) // (NC * NS * K)
E_PAD = NC * NS * S * K
ACC_ROWS = 10112
ROWS_PER_SUB = ACC_ROWS // NS


def _sc_body(x_hbm, ewi_hbm, part_hbm, idx_v, rows_v, acc,
             sem_i0, sem_i1, sem_i2, sem_g0, sem_g1, sem_g2,
             sem_s0, sem_s1, sem_s2):
    c = lax.axis_index("c")
    s = lax.axis_index("s")
    sem_i = (sem_i0, sem_i1, sem_i2)
    sem_g = (sem_g0, sem_g1, sem_g2)
    sem_s = (sem_s0, sem_s1, sem_s2)

    zero16 = jnp.zeros((16,), jnp.float32)

    @plsc.parallel_loop(0, K, 1, unroll=2)
    def _zrow(i):
        for r in range(D_FEAT // 16):
            rows_v[0, i, pl.ds(r * 16, 16)] = zero16
    row0 = s * ROWS_PER_SUB
    off = 0
    while off < ROWS_PER_SUB:
        n = min(K, ROWS_PER_SUB - off)
        pltpu.sync_copy(rows_v.at[0, pl.ds(0, n)], acc.at[pl.ds(row0 + off, n)])
        off += n
    plsc.subcore_barrier()

    def _issue_idx(j, slot):
        pltpu.async_copy(ewi_hbm.at[c, s, j], idx_v.at[slot], sem_i[slot])

    def _wait_idx(slot):
        pltpu.make_async_copy(ewi_hbm.at[c, s, 0], idx_v.at[slot],
                              sem_i[slot]).wait()

    def _issue_gather(slot):
        pltpu.async_copy(x_hbm.at[idx_v.at[slot, 0]], rows_v.at[slot],
                         sem_g[slot])

    def _wait_gather(slot):
        pltpu.make_async_copy(x_hbm.at[idx_v.at[slot, 0]], rows_v.at[slot],
                              sem_g[slot]).wait()

    def _issue_scatter(slot):
        pltpu.async_copy(rows_v.at[slot], acc.at[idx_v.at[slot, 1]],
                         sem_s[slot], add=True)

    def _wait_scatter(slot):
        pltpu.make_async_copy(rows_v.at[slot], acc.at[idx_v.at[slot, 1]],
                              sem_s[slot]).wait()

    def _scale(slot):
        @plsc.parallel_loop(0, K // 16, 1, unroll=2)
        def _grp(g):
            wv16 = lax.bitcast_convert_type(
                idx_v[slot, 2, pl.ds(g * 16, 16)], jnp.float32)
            for l in range(16):
                wl = wv16[l]
                row = g * 16 + l
                for r in range(D_FEAT // 16):
                    sl = pl.ds(r * 16, 16)
                    rows_v[slot, row, sl] = rows_v[slot, row, sl] * wl

    pltpu.sync_copy(ewi_hbm.at[c, s, 0], idx_v.at[0])
    _issue_gather(0)
    _issue_idx(1, 1)

    def _tri(t, _):
        for k in range(3):
            a, b, n2 = k % 3, (k + 1) % 3, (k + 2) % 3
            j = 3 * t + k
            _wait_gather(a)

            @pl.when(j + 1 < S)
            def _():
                _wait_idx(b)
                _issue_gather(b)

            _scale(a)

            @pl.when(j >= 1)
            def _():
                _wait_scatter(n2)

            @pl.when(j + 2 < S)
            def _():
                _issue_idx(j + 2, n2)

            _issue_scatter(a)
        return 0

    lax.fori_loop(0, S // 3, _tri, 0)
    _wait_scatter((S - 1) % 3)
    plsc.subcore_barrier()

    pltpu.sync_copy(acc.at[pl.ds(row0, ROWS_PER_SUB)],
                    part_hbm.at[c, pl.ds(row0, ROWS_PER_SUB)])


_sc_spmm = pl.kernel(
    _sc_body,
    out_type=jax.ShapeDtypeStruct((NC, ACC_ROWS, D_FEAT), jnp.float32),
    mesh=plsc.VectorSubcoreMesh(core_axis_name="c", subcore_axis_name="s"),
    scratch_types=[
        pltpu.VMEM((3, 3, K), jnp.int32),
        pltpu.VMEM((3, K, D_FEAT), jnp.float32),
        pltpu.VMEM_SHARED((ACC_ROWS, D_FEAT), jnp.float32),
    ] + [pltpu.SemaphoreType.DMA] * 9,
)


def _add_body(p_ref, o_ref):
    o_ref[...] = p_ref[0] + p_ref[1]


_BLK = 1000
_tc_add = pl.pallas_call(
    _add_body,
    grid=(N_NODES // _BLK,),
    in_specs=[pl.BlockSpec((NC, _BLK, D_FEAT), lambda i: (0, i, 0))],
    out_specs=pl.BlockSpec((_BLK, D_FEAT), lambda i: (i, 0)),
    out_shape=jax.ShapeDtypeStruct((N_NODES, D_FEAT), jnp.float32),
)


def kernel(x, edge_index, edge_weight):
    src = edge_index[1].astype(jnp.int32)
    dst = edge_index[0].astype(jnp.int32)
    wbits = lax.bitcast_convert_type(edge_weight.astype(jnp.float32),
                                     jnp.int32)
    pad = E_PAD - N_EDGES
    ewi = jnp.stack([
        jnp.pad(src, (0, pad)).reshape(NC, NS, S, K),
        jnp.pad(dst, (0, pad)).reshape(NC, NS, S, K),
        jnp.pad(wbits, (0, pad)).reshape(NC, NS, S, K),
    ], axis=3)
    part = _sc_spmm(x, ewi)
    return _tc_add(part)


# speedup vs baseline: 1.0366x; 1.0366x over previous
import jax
import jax.numpy as jnp
from jax import lax
from jax.experimental import pallas as pl
from jax.experimental.pallas import tpu as pltpu
from jax.experimental.pallas import tpu_sc as plsc

N_NODES = 10000
D_FEAT = 128
N_EDGES = 320000

NC = 2
NS = 16
K = 112
S = (N_EDGES + NC * NS * K - 1) // (NC * NS * K)
E_PAD = NC * NS * S * K
ACC_ROWS = 10112
ROWS_PER_SUB = ACC_ROWS // NS


def _sc_body(x_hbm, ewi_hbm, part_hbm, idx_v, rows_v, acc,
             sem_i0, sem_i1, sem_i2, sem_g0, sem_g1, sem_g2,
             sem_s0, sem_s1, sem_s2):
    c = lax.axis_index("c")
    s = lax.axis_index("s")
    sem_i = (sem_i0, sem_i1, sem_i2)
    sem_g = (sem_g0, sem_g1, sem_g2)
    sem_s = (sem_s0, sem_s1, sem_s2)

    zero16 = jnp.zeros((16,), jnp.float32)

    @plsc.parallel_loop(0, K, 1, unroll=2)
    def _zrow(i):
        for r in range(D_FEAT // 16):
            rows_v[0, i, pl.ds(r * 16, 16)] = zero16
    row0 = s * ROWS_PER_SUB
    off = 0
    while off < ROWS_PER_SUB:
        n = min(K, ROWS_PER_SUB - off)
        pltpu.sync_copy(rows_v.at[0, pl.ds(0, n)], acc.at[pl.ds(row0 + off, n)])
        off += n
    plsc.subcore_barrier()

    def _issue_idx(j, slot):
        pltpu.async_copy(ewi_hbm.at[c, s, j], idx_v.at[slot], sem_i[slot])

    def _wait_idx(slot):
        pltpu.make_async_copy(ewi_hbm.at[c, s, 0], idx_v.at[slot],
                              sem_i[slot]).wait()

    def _issue_gather(slot):
        pltpu.async_copy(x_hbm.at[idx_v.at[slot, 0]], rows_v.at[slot],
                         sem_g[slot])

    def _wait_gather(slot):
        pltpu.make_async_copy(x_hbm.at[idx_v.at[slot, 0]], rows_v.at[slot],
                              sem_g[slot]).wait()

    def _issue_scatter(slot):
        pltpu.async_copy(rows_v.at[slot], acc.at[idx_v.at[slot, 1]],
                         sem_s[slot], add=True)

    def _wait_scatter(slot):
        pltpu.make_async_copy(rows_v.at[slot], acc.at[idx_v.at[slot, 1]],
                              sem_s[slot]).wait()

    def _scale(slot):
        @plsc.parallel_loop(0, K // 16, 1, unroll=2)
        def _grp(g):
            wv16 = lax.bitcast_convert_type(
                idx_v[slot, 2, pl.ds(g * 16, 16)], jnp.float32)
            for l in range(16):
                wl = wv16[l]
                row = g * 16 + l
                for r in range(D_FEAT // 16):
                    sl = pl.ds(r * 16, 16)
                    rows_v[slot, row, sl] = rows_v[slot, row, sl] * wl

    pltpu.sync_copy(ewi_hbm.at[c, s, 0], idx_v.at[0])
    _issue_gather(0)
    _issue_idx(1, 1)

    def _tri(t, _):
        for k in range(3):
            a, b, n2 = k % 3, (k + 1) % 3, (k + 2) % 3
            j = 3 * t + k

            @pl.when(j + 1 < S)
            def _():
                _wait_idx(b)
                _issue_gather(b)

            _wait_gather(a)
            _scale(a)
            _issue_scatter(a)

            @pl.when(j >= 1)
            def _():
                _wait_scatter(n2)

            @pl.when(j + 2 < S)
            def _():
                _issue_idx(j + 2, n2)
        return 0

    lax.fori_loop(0, S // 3, _tri, 0)
    _wait_scatter((S - 1) % 3)
    plsc.subcore_barrier()

    pltpu.sync_copy(acc.at[pl.ds(row0, ROWS_PER_SUB)],
                    part_hbm.at[c, pl.ds(row0, ROWS_PER_SUB)])


_sc_spmm = pl.kernel(
    _sc_body,
    out_type=jax.ShapeDtypeStruct((NC, ACC_ROWS, D_FEAT), jnp.float32),
    mesh=plsc.VectorSubcoreMesh(core_axis_name="c", subcore_axis_name="s"),
    scratch_types=[
        pltpu.VMEM((3, 3, K), jnp.int32),
        pltpu.VMEM((3, K, D_FEAT), jnp.float32),
        pltpu.VMEM_SHARED((ACC_ROWS, D_FEAT), jnp.float32),
    ] + [pltpu.SemaphoreType.DMA] * 9,
)


def _add_body(p_ref, o_ref):
    o_ref[...] = p_ref[0] + p_ref[1]


_BLK = 1000
_tc_add = pl.pallas_call(
    _add_body,
    grid=(N_NODES // _BLK,),
    in_specs=[pl.BlockSpec((NC, _BLK, D_FEAT), lambda i: (0, i, 0))],
    out_specs=pl.BlockSpec((_BLK, D_FEAT), lambda i: (i, 0)),
    out_shape=jax.ShapeDtypeStruct((N_NODES, D_FEAT), jnp.float32),
)


def kernel(x, edge_index, edge_weight):
    src = edge_index[1].astype(jnp.int32)
    dst = edge_index[0].astype(jnp.int32)
    wbits = lax.bitcast_convert_type(edge_weight.astype(jnp.float32),
                                     jnp.int32)
    pad = E_PAD - N_EDGES
    ewi = jnp.stack([
        jnp.pad(src, (0, pad)).reshape(NC, NS, S, K),
        jnp.pad(dst, (0, pad)).reshape(NC, NS, S, K),
        jnp.pad(wbits, (0, pad)).reshape(NC, NS, S, K),
    ], axis=3)
    part = _sc_spmm(x, ewi)
    return _tc_add(part)
